# Initial kernel scaffold; baseline (speedup 1.0000x reference)
import jax
import jax.numpy as jnp
from jax.experimental import pallas as pl


def kernel(image, flow):
    raise NotImplementedError("write your pallas kernel here")


# speedup vs baseline: 1.3415x; 1.3415x over previous
import functools

import jax
import jax.numpy as jnp
from jax import lax
from jax.experimental import pallas as pl
from jax.experimental.pallas import tpu as pltpu
from jax.experimental.pallas import tpu_sc as plsc


def _warp_sc(img_flat, fy_flat, fx_flat, *, B, H, W, C, Q):
    Nq = B * H * W
    info = plsc.get_sparse_core_info()
    NC, NS, L = info.num_cores, info.num_subcores, info.num_lanes
    NW = NC * NS
    assert H & (H - 1) == 0 and W & (W - 1) == 0, "H, W must be powers of two"
    assert C % L == 0
    assert Nq % (NW * Q) == 0 and Q % L == 0
    per_w = Nq // NW
    n_chunks = per_w // Q
    w_shift = W.bit_length() - 1
    hw = H * W

    mesh = plsc.VectorSubcoreMesh(core_axis_name="c", subcore_axis_name="s")

    @functools.partial(
        pl.kernel,
        mesh=mesh,
        compiler_params=pltpu.CompilerParams(use_tc_tiling_on_sc=False),
        out_type=jax.ShapeDtypeStruct((Nq, C), jnp.float32),
        scratch_types=[
            pltpu.VMEM((Q,), jnp.float32),
            pltpu.VMEM((Q,), jnp.float32),
            pltpu.VMEM((Q,), jnp.int32),
            pltpu.VMEM((Q,), jnp.int32),
            pltpu.VMEM((Q,), jnp.int32),
            pltpu.VMEM((Q,), jnp.int32),
            pltpu.VMEM((Q,), jnp.float32),
            pltpu.VMEM((Q,), jnp.float32),
            pltpu.VMEM((Q,), jnp.float32),
            pltpu.VMEM((Q,), jnp.float32),
            pltpu.VMEM((Q, C), jnp.float32),
            pltpu.VMEM((Q, C), jnp.float32),
            pltpu.VMEM((Q, C), jnp.float32),
            pltpu.VMEM((Q, C), jnp.float32),
            pltpu.VMEM((Q, C), jnp.float32),
            pltpu.SemaphoreType.DMA,
        ],
    )
    def warp(img, fyf, fxf, out, fy_v, fx_v, itl, itr, ibl, ibr,
             w00, w01, w10, w11, rtl, rtr, rbl, rbr, out_v, sem):
        wid = lax.axis_index("s") * NC + lax.axis_index("c")
        wbase = wid * per_w

        def chunk_body(ci, carry):
            base = wbase + ci * Q
            pltpu.sync_copy(fyf.at[pl.ds(base, Q)], fy_v)
            pltpu.sync_copy(fxf.at[pl.ds(base, Q)], fx_v)

            for g in range(Q // L):
                s = pl.ds(g * L, L)
                n = (base + g * L) + lax.iota(jnp.int32, L)
                y = (n >> w_shift) & (H - 1)
                x = n & (W - 1)
                qy = y.astype(jnp.float32) - fy_v[s]
                qx = x.astype(jnp.float32) - fx_v[s]
                fyi = jnp.clip(qy, 0.0, float(H - 2)).astype(jnp.int32)
                fxi = jnp.clip(qx, 0.0, float(W - 2)).astype(jnp.int32)
                ay = jnp.clip(qy - fyi.astype(jnp.float32), 0.0, 1.0)
                ax = jnp.clip(qx - fxi.astype(jnp.float32), 0.0, 1.0)
                boff = n & jnp.int32(-hw)
                tl = boff + (fyi << w_shift) + fxi
                itl[s] = tl
                itr[s] = tl + 1
                ibl[s] = tl + W
                ibr[s] = tl + (W + 1)
                by = 1.0 - ay
                bx = 1.0 - ax
                w00[s] = by * bx
                w01[s] = by * ax
                w10[s] = ay * bx
                w11[s] = ay * ax

            c0 = pltpu.async_copy(img.at[itl], rtl, sem)
            c1 = pltpu.async_copy(img.at[itr], rtr, sem)
            c2 = pltpu.async_copy(img.at[ibl], rbl, sem)
            c3 = pltpu.async_copy(img.at[ibr], rbr, sem)
            c0.wait()
            c1.wait()
            c2.wait()
            c3.wait()

            def gbody(g, carry2):
                b16 = g * L
                sw = pl.ds(b16, L)
                avec = w00[sw]
                bvec = w01[sw]
                cvec = w10[sw]
                dvec = w11[sw]
                for t in range(L):
                    i = b16 + t
                    a = jnp.full((L,), avec[t], jnp.float32)
                    b = jnp.full((L,), bvec[t], jnp.float32)
                    c = jnp.full((L,), cvec[t], jnp.float32)
                    d = jnp.full((L,), dvec[t], jnp.float32)
                    for j in range(C // L):
                        sj = pl.ds(j * L, L)
                        out_v[i, sj] = (a * rtl[i, sj] + b * rtr[i, sj]
                                        + c * rbl[i, sj] + d * rbr[i, sj])
                return carry2

            lax.fori_loop(0, Q // L, gbody, 0)
            pltpu.sync_copy(out_v, out.at[pl.ds(base, Q)])
            return carry

        lax.fori_loop(0, n_chunks, chunk_body, 0)

    return warp(img_flat, fy_flat, fx_flat)


def kernel(image, flow):
    B, H, W, C = image.shape
    img_flat = image.reshape(B * H * W, C)
    fy = flow[..., 0].reshape(-1)
    fx = flow[..., 1].reshape(-1)
    out = _warp_sc(img_flat, fy, fx, B=B, H=H, W=W, C=C, Q=128)
    return out.reshape(B, H, W, C)


# speedup vs baseline: 1.9088x; 1.4229x over previous
import functools

import jax
import jax.numpy as jnp
from jax import lax
from jax.experimental import pallas as pl
from jax.experimental.pallas import tpu as pltpu
from jax.experimental.pallas import tpu_sc as plsc


def _warp_sc(img_flat, fy_flat, fx_flat, *, B, H, W, C, Q):
    Nq = B * H * W
    info = plsc.get_sparse_core_info()
    NC, NS, L = info.num_cores, info.num_subcores, info.num_lanes
    NW = NC * NS
    assert H & (H - 1) == 0 and W & (W - 1) == 0, "H, W must be powers of two"
    assert C % L == 0
    assert Nq % (NW * Q) == 0 and Q % L == 0
    per_w = Nq // NW
    n_chunks = per_w // Q
    assert n_chunks % 2 == 0
    w_shift = W.bit_length() - 1
    hw = H * W

    mesh = plsc.VectorSubcoreMesh(core_axis_name="c", subcore_axis_name="s")

    def slot_scratch():
        return (
            [pltpu.VMEM((Q,), jnp.int32) for _ in range(4)]
            + [pltpu.VMEM((Q,), jnp.float32) for _ in range(2)]
            + [pltpu.VMEM((Q, C), jnp.float32) for _ in range(4)]
            + [pltpu.SemaphoreType.DMA]
        )

    @functools.partial(
        pl.kernel,
        mesh=mesh,
        compiler_params=pltpu.CompilerParams(use_tc_tiling_on_sc=False),
        out_type=jax.ShapeDtypeStruct((Nq, C), jnp.float32),
        scratch_types=(
            [pltpu.VMEM((Q,), jnp.float32) for _ in range(2)]
            + [pltpu.VMEM((Q, C), jnp.float32)]
            + slot_scratch() + slot_scratch()
        ),
    )
    def warp(img, fyf, fxf, out,
             fy_v, fx_v, out_v,
             itl0, itr0, ibl0, ibr0, ay0, ax0, rtl0, rtr0, rbl0, rbr0, sem0,
             itl1, itr1, ibl1, ibr1, ay1, ax1, rtl1, rtr1, rbl1, rbr1, sem1):
        wid = lax.axis_index("s") * NC + lax.axis_index("c")
        wbase = wid * per_w
        slots = (
            (itl0, itr0, ibl0, ibr0, ay0, ax0, rtl0, rtr0, rbl0, rbr0, sem0),
            (itl1, itr1, ibl1, ibr1, ay1, ax1, rtl1, rtr1, rbl1, rbr1, sem1),
        )

        def prep(ci, slot):
            itl, itr, ibl, ibr, ayb, axb, rtl, rtr, rbl, rbr, sem = slots[slot]
            base = wbase + ci * Q
            pltpu.sync_copy(fyf.at[pl.ds(base, Q)], fy_v)
            pltpu.sync_copy(fxf.at[pl.ds(base, Q)], fx_v)
            for g in range(Q // L):
                s = pl.ds(g * L, L)
                n = (base + g * L) + lax.iota(jnp.int32, L)
                y = (n >> w_shift) & (H - 1)
                x = n & (W - 1)
                qy = y.astype(jnp.float32) - fy_v[s]
                qx = x.astype(jnp.float32) - fx_v[s]
                fyi = jnp.clip(qy, 0.0, float(H - 2)).astype(jnp.int32)
                fxi = jnp.clip(qx, 0.0, float(W - 2)).astype(jnp.int32)
                ayb[s] = jnp.clip(qy - fyi.astype(jnp.float32), 0.0, 1.0)
                axb[s] = jnp.clip(qx - fxi.astype(jnp.float32), 0.0, 1.0)
                boff = n & jnp.int32(-hw)
                tl = boff + (fyi << w_shift) + fxi
                itl[s] = tl
                itr[s] = tl + 1
                ibl[s] = tl + W
                ibr[s] = tl + (W + 1)
            pltpu.async_copy(img.at[itl], rtl, sem)
            pltpu.async_copy(img.at[itr], rtr, sem)
            pltpu.async_copy(img.at[ibl], rbl, sem)
            pltpu.async_copy(img.at[ibr], rbr, sem)

        def finish(ci, slot):
            itl, itr, ibl, ibr, ayb, axb, rtl, rtr, rbl, rbr, sem = slots[slot]
            base = wbase + ci * Q
            for rows in (rtl, rtr, rbl, rbr):
                pltpu.make_async_copy(img.at[itl], rows, sem).wait()

            def gbody(g, carry):
                b16 = g * L
                sw = pl.ds(b16, L)
                ayv = ayb[sw]
                axv = axb[sw]
                for t in range(L):
                    i = b16 + t
                    ay = jnp.full((L,), ayv[t], jnp.float32)
                    ax = jnp.full((L,), axv[t], jnp.float32)
                    for j in range(C // L):
                        sj = pl.ds(j * L, L)
                        tlv = rtl[i, sj]
                        trv = rtr[i, sj]
                        blv = rbl[i, sj]
                        brv = rbr[i, sj]
                        top = tlv + ax * (trv - tlv)
                        bot = blv + ax * (brv - blv)
                        out_v[i, sj] = top + ay * (bot - top)
                return carry

            lax.fori_loop(0, Q // L, gbody, 0)
            pltpu.sync_copy(out_v, out.at[pl.ds(base, Q)])

        prep(0, 0)

        def body(k, carry):
            ci = 2 * k
            prep(ci + 1, 1)
            finish(ci, 0)

            @pl.when(ci + 2 < n_chunks)
            def _():
                prep(ci + 2, 0)

            finish(ci + 1, 1)
            return carry

        lax.fori_loop(0, n_chunks // 2, body, 0)

    return warp(img_flat, fy_flat, fx_flat)


def kernel(image, flow):
    B, H, W, C = image.shape
    img_flat = image.reshape(B * H * W, C)
    fy = flow[..., 0].reshape(-1)
    fx = flow[..., 1].reshape(-1)
    out = _warp_sc(img_flat, fy, fx, B=B, H=H, W=W, C=C, Q=128)
    return out.reshape(B, H, W, C)
